# baseline (speedup 1.0000x reference)
import functools

import jax
import jax.numpy as jnp
from jax import lax
from jax.experimental import pallas as pl
from jax.experimental.pallas import tpu as pltpu
from jax.experimental.pallas import tpu_sc as plsc

_EPS = 1e-6
_L = 16
_NC = 2
_NS = 16
_NW = _NC * _NS


def _softplus16(s):
    t = jnp.exp(-jnp.abs(s))
    c = 1.0 + t
    y = t * (0.9991150 + t * (-0.4899597 + t * 0.1560245))
    y = y - 1.0 + c * jnp.exp(-y)
    y = y - 1.0 + c * jnp.exp(-y)
    return jnp.maximum(s, 0.0) + y


def kernel(x, attr, mus, sigmas):
    B, D = x.shape
    A = mus.shape[0]
    G = D // _L
    rows_w = B // _NW
    CH = min(128, rows_w)
    nch = rows_w // CH

    mesh = plsc.VectorSubcoreMesh(core_axis_name="c", subcore_axis_name="s")

    @functools.partial(
        pl.kernel,
        out_type=jax.ShapeDtypeStruct((B, D), jnp.float32),
        mesh=mesh,
        compiler_params=pltpu.CompilerParams(needs_layout_passes=False),
        scratch_types=[
            pltpu.VMEM((A, D), jnp.float32),
            pltpu.VMEM((A, D), jnp.float32),
            pltpu.VMEM((A, D), jnp.float32),
            pltpu.VMEM((A, D), jnp.float32),
            pltpu.VMEM((2, CH, D), jnp.float32),
            pltpu.VMEM((2, CH, D), jnp.float32),
            pltpu.VMEM((2, CH), jnp.int32),
            pltpu.SemaphoreType.DMA,
            pltpu.SemaphoreType.DMA,
            pltpu.SemaphoreType.DMA,
            pltpu.SemaphoreType.DMA,
        ],
    )
    def sc_kernel(x_hbm, attr_hbm, mus_hbm, sig_hbm, out_hbm,
                  mus_v, sig_v, scale_v, bias_v, xb2, ob2, ab2,
                  isem0, isem1, osem0, osem1):
        isems = (isem0, isem1)
        osems = (osem0, osem1)
        wid = lax.axis_index("s") * _NC + lax.axis_index("c")
        base = wid * rows_w

        def start_in(t):
            b = t % 2
            r0 = base + t * CH
            dx = pltpu.async_copy(x_hbm.at[pl.ds(r0, CH), :], xb2.at[b], isems[b])
            da = pltpu.async_copy(attr_hbm.at[pl.ds(r0, CH)], ab2.at[b], isems[b])
            return (dx, da)

        in_desc = {0: start_in(0)}

        pltpu.sync_copy(mus_hbm, mus_v)
        pltpu.sync_copy(sig_hbm, sig_v)

        for r in range(A):
            for g in range(G):
                s = sig_v[r, pl.ds(g * _L, _L)]
                m = mus_v[r, pl.ds(g * _L, _L)]
                sc = 1.0 / (_softplus16(s) + _EPS)
                scale_v[r, pl.ds(g * _L, _L)] = sc
                bias_v[r, pl.ds(g * _L, _L)] = -m * sc

        out_desc = {}
        for t in range(nch):
            b = t % 2
            if t + 1 < nch:
                in_desc[t + 1] = start_in(t + 1)
            for d in in_desc.pop(t):
                d.wait()
            if t - 2 in out_desc:
                out_desc.pop(t - 2).wait()
            xb, ob, ab = xb2.at[b], ob2.at[b], ab2.at[b]

            def row_body(jg, carry):
                av = ab[pl.ds(jg * _L, _L)]
                for l in range(_L):
                    j = jg * _L + l
                    a = av[l]
                    for g in range(G):
                        sl = pl.ds(g * _L, _L)
                        ob[j, sl] = xb[j, sl] * scale_v[a, sl] + bias_v[a, sl]
                return carry

            lax.fori_loop(0, CH // _L, row_body, 0)
            r0 = base + t * CH
            out_desc[t] = pltpu.async_copy(
                ob, out_hbm.at[pl.ds(r0, CH), :], osems[b])
        for t in sorted(out_desc):
            out_desc.pop(t).wait()

    return sc_kernel(x, attr, mus, sigmas)


# speedup vs baseline: 1.4508x; 1.4508x over previous
import functools

import jax
import jax.numpy as jnp
from jax import lax
from jax.experimental import pallas as pl
from jax.experimental.pallas import tpu as pltpu
from jax.experimental.pallas import tpu_sc as plsc

_EPS = 1e-6
_L = 16
_NC = 2
_NS = 16
_NW = _NC * _NS


def _softplus16(s):
    t = jnp.exp(-jnp.abs(s))
    c = 1.0 + t
    y = t * (0.9991150 + t * (-0.4899597 + t * 0.1560245))
    y = y - 1.0 + c * jnp.exp(-y)
    y = y - 1.0 + c * jnp.exp(-y)
    return jnp.maximum(s, 0.0) + y


def kernel(x, attr, mus, sigmas):
    B, D = x.shape
    A = mus.shape[0]
    G = D // _L
    rows_w = B // _NW
    CH = min(128, rows_w)
    nch = rows_w // CH

    mesh = plsc.VectorSubcoreMesh(core_axis_name="c", subcore_axis_name="s")

    @functools.partial(
        pl.kernel,
        out_type=jax.ShapeDtypeStruct((B, D), jnp.float32),
        mesh=mesh,
        compiler_params=pltpu.CompilerParams(needs_layout_passes=False),
        scratch_types=[
            pltpu.VMEM((A, D), jnp.float32),
            pltpu.VMEM((A, D), jnp.float32),
            pltpu.VMEM((A, D), jnp.float32),
            pltpu.VMEM((A, D), jnp.float32),
            pltpu.VMEM((2, CH, D), jnp.float32),
            pltpu.VMEM((2, CH, D), jnp.float32),
            pltpu.VMEM((2, CH), jnp.int32),
            pltpu.SemaphoreType.DMA,
            pltpu.SemaphoreType.DMA,
            pltpu.SemaphoreType.DMA,
            pltpu.SemaphoreType.DMA,
        ],
    )
    def sc_kernel(x_hbm, attr_hbm, mus_hbm, sig_hbm, out_hbm,
                  mus_v, sig_v, scale_v, bias_v, xb2, ob2, ab2,
                  isem0, isem1, osem0, osem1):
        isems = (isem0, isem1)
        osems = (osem0, osem1)
        wid = lax.axis_index("s") * _NC + lax.axis_index("c")
        base = wid * rows_w

        def start_in(t):
            b = t % 2
            r0 = base + t * CH
            dx = pltpu.async_copy(x_hbm.at[pl.ds(r0, CH), :], xb2.at[b], isems[b])
            da = pltpu.async_copy(attr_hbm.at[pl.ds(r0, CH)], ab2.at[b], isems[b])
            return (dx, da)

        in_desc = {0: start_in(0)}

        pltpu.sync_copy(mus_hbm, mus_v)
        pltpu.sync_copy(sig_hbm, sig_v)

        for r in range(A):
            for g in range(G):
                s = sig_v[r, pl.ds(g * _L, _L)]
                m = mus_v[r, pl.ds(g * _L, _L)]
                sc = 1.0 / (_softplus16(s) + _EPS)
                scale_v[r, pl.ds(g * _L, _L)] = sc
                bias_v[r, pl.ds(g * _L, _L)] = -m * sc

        out_desc = {}
        for t in range(nch):
            b = t % 2
            if t + 1 < nch:
                in_desc[t + 1] = start_in(t + 1)
            for d in in_desc.pop(t):
                d.wait()
            if t - 2 in out_desc:
                out_desc.pop(t - 2).wait()
            xb, ob, ab = xb2.at[b], ob2.at[b], ab2.at[b]

            def row_body(jg, carry):
                av = ab[pl.ds(jg * _L, _L)]
                for l in range(_L):
                    j = jg * _L + l
                    a = av[l]
                    sls = [pl.ds(g * _L, _L) for g in range(G)]
                    xs = [xb[j, sl] for sl in sls]
                    scs = [scale_v[a, sl] for sl in sls]
                    bss = [bias_v[a, sl] for sl in sls]
                    for g, sl in enumerate(sls):
                        ob[j, sl] = xs[g] * scs[g] + bss[g]
                return carry

            lax.fori_loop(0, CH // _L, row_body, 0)
            r0 = base + t * CH
            out_desc[t] = pltpu.async_copy(
                ob, out_hbm.at[pl.ds(r0, CH), :], osems[b])
        for t in sorted(out_desc):
            out_desc.pop(t).wait()

    return sc_kernel(x, attr, mus, sigmas)


# speedup vs baseline: 1.5087x; 1.0399x over previous
import functools

import jax
import jax.numpy as jnp
from jax import lax
from jax.experimental import pallas as pl
from jax.experimental.pallas import tpu as pltpu
from jax.experimental.pallas import tpu_sc as plsc

_EPS = 1e-6
_L = 16
_NC = 2
_NS = 16
_NW = _NC * _NS


def _softplus16(s):
    t = jnp.exp(-jnp.abs(s))
    c = 1.0 + t
    y = t * (0.9991150 + t * (-0.4899597 + t * 0.1560245))
    y = y - 1.0 + c * jnp.exp(-y)
    y = y - 1.0 + c * jnp.exp(-y)
    return jnp.maximum(s, 0.0) + y


def kernel(x, attr, mus, sigmas):
    B, D = x.shape
    A = mus.shape[0]
    G = D // _L
    rows_w = B // _NW
    CH = min(128, rows_w)
    nch = rows_w // CH

    mesh = plsc.VectorSubcoreMesh(core_axis_name="c", subcore_axis_name="s")

    @functools.partial(
        pl.kernel,
        out_type=jax.ShapeDtypeStruct((B, D), jnp.float32),
        mesh=mesh,
        compiler_params=pltpu.CompilerParams(needs_layout_passes=False),
        scratch_types=[
            pltpu.VMEM((A, D), jnp.float32),
            pltpu.VMEM((A, D), jnp.float32),
            pltpu.VMEM((A, D), jnp.int32),
            pltpu.VMEM((2, CH, D), jnp.float32),
            pltpu.VMEM((2, CH, D), jnp.float32),
            pltpu.VMEM((2, CH), jnp.int32),
            pltpu.SemaphoreType.DMA,
            pltpu.SemaphoreType.DMA,
            pltpu.SemaphoreType.DMA,
            pltpu.SemaphoreType.DMA,
        ],
    )
    def sc_kernel(x_hbm, attr_hbm, mus_hbm, sig_hbm, out_hbm,
                  mus_v, sig_v, ptab_v, xb2, ob2, ab2,
                  isem0, isem1, osem0, osem1):
        isems = (isem0, isem1)
        osems = (osem0, osem1)
        wid = lax.axis_index("s") * _NC + lax.axis_index("c")
        base = wid * rows_w

        def start_in(t):
            b = t % 2
            r0 = base + t * CH
            dx = pltpu.async_copy(x_hbm.at[pl.ds(r0, CH), :], xb2.at[b], isems[b])
            da = pltpu.async_copy(attr_hbm.at[pl.ds(r0, CH)], ab2.at[b], isems[b])
            return (dx, da)

        in_desc = {0: start_in(0)}

        pltpu.sync_copy(mus_hbm, mus_v)
        pltpu.sync_copy(sig_hbm, sig_v)

        for r in range(A):
            for g in range(G):
                s = sig_v[r, pl.ds(g * _L, _L)]
                m = mus_v[r, pl.ds(g * _L, _L)]
                sc = 1.0 / (_softplus16(s) + _EPS)
                pk = plsc.pack(sc, -m * sc, format=plsc.PackFormat.INTERLEAVED)
                ptab_v[r, pl.ds(g * _L, _L)] = plsc.bitcast(pk, jnp.int32)

        out_desc = {}
        for t in range(nch):
            b = t % 2
            if t + 1 < nch:
                in_desc[t + 1] = start_in(t + 1)
            for d in in_desc.pop(t):
                d.wait()
            if t - 2 in out_desc:
                out_desc.pop(t - 2).wait()
            xb, ob, ab = xb2.at[b], ob2.at[b], ab2.at[b]

            def row_body(jg, carry):
                av = ab[pl.ds(jg * _L, _L)]
                for l in range(_L):
                    j = jg * _L + l
                    a = av[l]
                    sls = [pl.ds(g * _L, _L) for g in range(G)]
                    xs = [xb[j, sl] for sl in sls]
                    pks = [plsc.unpack(
                        plsc.bitcast(ptab_v[a, sl], jnp.bfloat16),
                        format=plsc.PackFormat.INTERLEAVED) for sl in sls]
                    for g, sl in enumerate(sls):
                        scv, bsv = pks[g]
                        ob[j, sl] = xs[g] * scv + bsv
                return carry

            lax.fori_loop(0, CH // _L, row_body, 0)
            r0 = base + t * CH
            out_desc[t] = pltpu.async_copy(
                ob, out_hbm.at[pl.ds(r0, CH), :], osems[b])
        for t in sorted(out_desc):
            out_desc.pop(t).wait()

    return sc_kernel(x, attr, mus, sigmas)


# speedup vs baseline: 1.5750x; 1.0440x over previous
import functools

import jax
import jax.numpy as jnp
from jax import lax
from jax.experimental import pallas as pl
from jax.experimental.pallas import tpu as pltpu
from jax.experimental.pallas import tpu_sc as plsc

_EPS = 1e-6
_L = 16
_NC = 2
_NS = 16
_NW = _NC * _NS


def _softplus16(s):
    t = jnp.exp(-jnp.abs(s))
    c = 1.0 + t
    y = t * (0.9991150 + t * (-0.4899597 + t * 0.1560245))
    y = y - 1.0 + c * jnp.exp(-y)
    y = y - 1.0 + c * jnp.exp(-y)
    return jnp.maximum(s, 0.0) + y


def kernel(x, attr, mus, sigmas):
    B, D = x.shape
    A = mus.shape[0]
    G = D // _L
    rows_w = B // _NW
    CH = min(128, rows_w)
    nch = rows_w // CH

    mesh = plsc.VectorSubcoreMesh(core_axis_name="c", subcore_axis_name="s")

    @functools.partial(
        pl.kernel,
        out_type=jax.ShapeDtypeStruct((B, D), jnp.float32),
        mesh=mesh,
        compiler_params=pltpu.CompilerParams(needs_layout_passes=False),
        scratch_types=[
            pltpu.VMEM((A, D), jnp.float32),
            pltpu.VMEM((A, D), jnp.float32),
            pltpu.VMEM((A, D), jnp.int32),
            pltpu.VMEM((2, CH, D), jnp.float32),
            pltpu.VMEM((2, CH, D), jnp.float32),
            pltpu.VMEM((2, CH), jnp.int32),
            pltpu.SemaphoreType.DMA,
            pltpu.SemaphoreType.DMA,
            pltpu.SemaphoreType.DMA,
            pltpu.SemaphoreType.DMA,
        ],
    )
    def sc_kernel(x_hbm, attr_hbm, mus_hbm, sig_hbm, out_hbm,
                  mus_v, sig_v, ptab_v, xb2, ob2, ab2,
                  isem0, isem1, osem0, osem1):
        isems = (isem0, isem1)
        osems = (osem0, osem1)
        wid = lax.axis_index("s") * _NC + lax.axis_index("c")
        base = wid * rows_w

        def start_in(t):
            b = t % 2
            r0 = base + t * CH
            dx = pltpu.async_copy(x_hbm.at[pl.ds(r0, CH), :], xb2.at[b], isems[b])
            da = pltpu.async_copy(attr_hbm.at[pl.ds(r0, CH)], ab2.at[b], isems[b])
            return (dx, da)

        in_desc = {0: start_in(0)}

        pltpu.sync_copy(mus_hbm, mus_v)
        pltpu.sync_copy(sig_hbm, sig_v)

        for r in range(A):
            for g in range(G):
                s = sig_v[r, pl.ds(g * _L, _L)]
                m = mus_v[r, pl.ds(g * _L, _L)]
                sc = 1.0 / (_softplus16(s) + _EPS)
                pk = plsc.pack(sc, -m * sc, format=plsc.PackFormat.INTERLEAVED)
                ptab_v[r, pl.ds(g * _L, _L)] = plsc.bitcast(pk, jnp.int32)

        in_desc[1] = start_in(1)

        def chunk_pair(t2, carry):
            for b in range(2):
                t = t2 * 2 + b
                xb, ob, ab = xb2.at[b], ob2.at[b], ab2.at[b]
                r0 = base + t * CH
                pltpu.make_async_copy(
                    x_hbm.at[pl.ds(r0, CH), :], xb, isems[b]).wait()
                pltpu.make_async_copy(
                    attr_hbm.at[pl.ds(r0, CH)], ab, isems[b]).wait()

                @pl.when(t2 > 0)
                def _drain():
                    rp = base + (t - 2) * CH
                    pltpu.make_async_copy(
                        ob, out_hbm.at[pl.ds(rp, CH), :], osems[b]).wait()

                compute_chunk(xb, ob, ab)

                @pl.when(t + 2 < nch)
                def _prefetch():
                    rn = base + (t + 2) * CH
                    pltpu.async_copy(
                        x_hbm.at[pl.ds(rn, CH), :], xb, isems[b])
                    pltpu.async_copy(
                        attr_hbm.at[pl.ds(rn, CH)], ab, isems[b])

                pltpu.async_copy(ob, out_hbm.at[pl.ds(r0, CH), :], osems[b])
            return carry

        def compute_chunk(xb, ob, ab):
            def row_body(jg, carry):
                av = ab[pl.ds(jg * _L, _L)]
                for l in range(_L):
                    j = jg * _L + l
                    a = av[l]
                    sls = [pl.ds(g * _L, _L) for g in range(G)]
                    xs = [xb[j, sl] for sl in sls]
                    pks = [plsc.unpack(
                        plsc.bitcast(ptab_v[a, sl], jnp.bfloat16),
                        format=plsc.PackFormat.INTERLEAVED) for sl in sls]
                    for g, sl in enumerate(sls):
                        scv, bsv = pks[g]
                        ob[j, sl] = xs[g] * scv + bsv
                return carry

            lax.fori_loop(0, CH // _L, row_body, 0)

        lax.fori_loop(0, nch // 2, chunk_pair, 0)
        for b in range(2):
            t = nch - 2 + b
            pltpu.make_async_copy(
                ob2.at[b], out_hbm.at[pl.ds(base + t * CH, CH), :],
                osems[b]).wait()

    return sc_kernel(x, attr, mus, sigmas)


# speedup vs baseline: 1.8530x; 1.1765x over previous
import functools

import jax
import jax.numpy as jnp
from jax import lax
from jax.experimental import pallas as pl
from jax.experimental.pallas import tpu as pltpu
from jax.experimental.pallas import tpu_sc as plsc

_EPS = 1e-6
_L = 16
_NC = 2
_NS = 16
_NW = _NC * _NS


def _softplus16(s):
    t = jnp.exp(-jnp.abs(s))
    c = 1.0 + t
    y = t * (0.9991150 + t * (-0.4899597 + t * 0.1560245))
    y = y - 1.0 + c * jnp.exp(-y)
    y = y - 1.0 + c * jnp.exp(-y)
    return jnp.maximum(s, 0.0) + y


def kernel(x, attr, mus, sigmas):
    B, D = x.shape
    A = mus.shape[0]
    G = D // _L
    rows_w = B // _NW
    CH = min(128, rows_w)
    nch = rows_w // CH

    mesh = plsc.VectorSubcoreMesh(core_axis_name="c", subcore_axis_name="s")

    @functools.partial(
        pl.kernel,
        out_type=jax.ShapeDtypeStruct((B, D), jnp.float32),
        mesh=mesh,
        compiler_params=pltpu.CompilerParams(needs_layout_passes=False),
        scratch_types=[
            pltpu.VMEM((A, D), jnp.float32),
            pltpu.VMEM((A, D), jnp.float32),
            pltpu.VMEM((A, D), jnp.int32),
            pltpu.VMEM((2, CH, D), jnp.float32),
            pltpu.VMEM((2, CH, D), jnp.float32),
            pltpu.VMEM((CH + _L,), jnp.int32),
            pltpu.VMEM((CH + _L,), jnp.int32),
            pltpu.SemaphoreType.DMA,
            pltpu.SemaphoreType.DMA,
            pltpu.SemaphoreType.DMA,
            pltpu.SemaphoreType.DMA,
        ],
    )
    def sc_kernel(x_hbm, attr_hbm, mus_hbm, sig_hbm, out_hbm,
                  mus_v, sig_v, ptab_v, xb2, ob2, ab0, ab1,
                  isem0, isem1, osem0, osem1):
        abufs = (ab0, ab1)
        isems = (isem0, isem1)
        osems = (osem0, osem1)
        wid = lax.axis_index("s") * _NC + lax.axis_index("c")
        base = wid * rows_w

        def start_in(t):
            b = t % 2
            r0 = base + t * CH
            dx = pltpu.async_copy(x_hbm.at[pl.ds(r0, CH), :], xb2.at[b], isems[b])
            da = pltpu.async_copy(attr_hbm.at[pl.ds(r0, CH)],
                                  abufs[b].at[pl.ds(0, CH)], isems[b])
            return (dx, da)

        in_desc = {0: start_in(0)}

        pltpu.sync_copy(mus_hbm, mus_v)
        pltpu.sync_copy(sig_hbm, sig_v)

        def build_row(r, carry):
            for g in range(G):
                s = sig_v[r, pl.ds(g * _L, _L)]
                m = mus_v[r, pl.ds(g * _L, _L)]
                sc = 1.0 / (_softplus16(s) + _EPS)
                pk = plsc.pack(sc, -m * sc, format=plsc.PackFormat.INTERLEAVED)
                ptab_v[r, pl.ds(g * _L, _L)] = plsc.bitcast(pk, jnp.int32)
            return carry

        lax.fori_loop(0, A, build_row, 0)

        in_desc[1] = start_in(1)

        def chunk_pair(t2, carry):
            for b in range(2):
                t = t2 * 2 + b
                xb, ob, ab = xb2.at[b], ob2.at[b], abufs[b]
                r0 = base + t * CH
                pltpu.make_async_copy(
                    x_hbm.at[pl.ds(r0, CH), :], xb, isems[b]).wait()
                pltpu.make_async_copy(
                    attr_hbm.at[pl.ds(r0, CH)], abufs[b].at[pl.ds(0, CH)],
                    isems[b]).wait()

                @pl.when(t2 > 0)
                def _drain():
                    rp = base + (t - 2) * CH
                    pltpu.make_async_copy(
                        ob, out_hbm.at[pl.ds(rp, CH), :], osems[b]).wait()

                compute_chunk(xb, ob, ab)

                @pl.when(t + 2 < nch)
                def _prefetch():
                    rn = base + (t + 2) * CH
                    pltpu.async_copy(
                        x_hbm.at[pl.ds(rn, CH), :], xb, isems[b])
                    pltpu.async_copy(
                        attr_hbm.at[pl.ds(rn, CH)],
                        abufs[b].at[pl.ds(0, CH)], isems[b])

                pltpu.async_copy(ob, out_hbm.at[pl.ds(r0, CH), :], osems[b])
            return carry

        RB = 8

        def compute_chunk(xb, ob, ab):
            def row_body(jg, carry):
                av = ab[pl.ds(jg * RB, _L)]
                for l in range(RB):
                    j = jg * RB + l
                    a = av[l]
                    sls = [pl.ds(g * _L, _L) for g in range(G)]
                    xs = [xb[j, sl] for sl in sls]
                    pks = [plsc.unpack(
                        plsc.bitcast(ptab_v[a, sl], jnp.bfloat16),
                        format=plsc.PackFormat.INTERLEAVED) for sl in sls]
                    for g, sl in enumerate(sls):
                        scv, bsv = pks[g]
                        ob[j, sl] = xs[g] * scv + bsv
                return carry

            lax.fori_loop(0, CH // RB, row_body, 0)

        lax.fori_loop(0, nch // 2, chunk_pair, 0)
        for b in range(2):
            t = nch - 2 + b
            pltpu.make_async_copy(
                ob2.at[b], out_hbm.at[pl.ds(base + t * CH, CH), :],
                osems[b]).wait()

    return sc_kernel(x, attr, mus, sigmas)
